# Initial kernel scaffold; baseline (speedup 1.0000x reference)
import jax
import jax.numpy as jnp
from jax.experimental import pallas as pl


def kernel(features, vals0, vals1, kernel, bias, rows0, cols0, rows1, cols1):
    raise NotImplementedError("write your pallas kernel here")


# speedup vs baseline: 6.0977x; 6.0977x over previous
import functools

import jax
import jax.numpy as jnp
from jax import lax
from jax.experimental import pallas as pl
from jax.experimental.pallas import tpu as pltpu
from jax.experimental.pallas import tpu_sc as plsc

_CH = 128


def _sc_spmm(featT, vals, rows, cols):
    N, B = featT.shape
    S, NNZ = vals.shape
    n_sub = 16
    ept = NNZ // n_sub
    nchunks = ept // _CH
    rows_per_sub = N // n_sub

    mesh = plsc.VectorSubcoreMesh(core_axis_name="c", subcore_axis_name="s")

    @functools.partial(
        pl.kernel,
        mesh=mesh,
        compiler_params=pltpu.CompilerParams(use_tc_tiling_on_sc=False),
        out_type=jax.ShapeDtypeStruct((S, N, B), jnp.float32),
        scratch_types=[
            pltpu.VMEM((_CH, B), jnp.float32),
            pltpu.VMEM((_CH,), jnp.int32),
            pltpu.VMEM((_CH,), jnp.int32),
            pltpu.VMEM((_CH,), jnp.float32),
            pltpu.VMEM_SHARED((N, B), jnp.float32),
            pltpu.SemaphoreType.DMA,
        ],
    )
    def k(featT_h, vals_h, rows_h, cols_h, out_h, gbuf, colv, rowv, valv, acc, sem):
        c = lax.axis_index("c")
        s = lax.axis_index("s")

        def zrow(r, carry):
            for q in range(B // 16):
                gbuf[r, pl.ds(q * 16, 16)] = jnp.zeros((16,), jnp.float32)
            return carry

        lax.fori_loop(0, _CH, zrow, None)
        for kk in range(rows_per_sub // _CH):
            pltpu.sync_copy(gbuf, acc.at[pl.ds(s * rows_per_sub + kk * _CH, _CH)])
        plsc.subcore_barrier()

        def chunk(it, carry):
            base = s * ept + it * _CH
            pltpu.sync_copy(cols_h.at[c, pl.ds(base, _CH)], colv)
            pltpu.sync_copy(rows_h.at[c, pl.ds(base, _CH)], rowv)
            pltpu.sync_copy(vals_h.at[c, pl.ds(base, _CH)], valv)
            pltpu.async_copy(featT_h.at[colv], gbuf, sem).wait()
            for j in range(_CH // 16):
                va = valv[pl.ds(j * 16, 16)]
                for e in range(16):
                    r = j * 16 + e
                    v = va[e]
                    for q in range(B // 16):
                        sl = pl.ds(q * 16, 16)
                        gbuf[r, sl] = gbuf[r, sl] * v
            pltpu.sync_copy(gbuf, acc.at[rowv], add=True)
            return carry

        lax.fori_loop(0, nchunks, chunk, None)
        plsc.subcore_barrier()
        pltpu.sync_copy(
            acc.at[pl.ds(s * rows_per_sub, rows_per_sub)],
            out_h.at[c, pl.ds(s * rows_per_sub, rows_per_sub)],
        )

    return k(featT, vals, rows, cols)


def _combine(partials, weights, bias):
    S, N, B = partials.shape
    BN = 512

    def body(p_ref, k_ref, b_ref, o_ref):
        x = p_ref[0] * k_ref[0, 0] + p_ref[1] * k_ref[1, 0] + b_ref[0]
        o_ref[...] = jnp.clip(x, 0.0, 1.0).T

    return pl.pallas_call(
        body,
        grid=(N // BN,),
        in_specs=[
            pl.BlockSpec((S, BN, B), lambda i: (0, i, 0)),
            pl.BlockSpec(memory_space=pltpu.SMEM),
            pl.BlockSpec(memory_space=pltpu.SMEM),
        ],
        out_specs=pl.BlockSpec((B, BN), lambda i: (0, i)),
        out_shape=jax.ShapeDtypeStruct((B, N), jnp.float32),
    )(partials, weights, bias)


def kernel(features, vals0, vals1, kernel, bias, rows0, cols0, rows1, cols1):
    featT = features.T
    vals = jnp.stack([vals0, vals1])
    rows = jnp.stack([rows0, rows1])
    cols = jnp.stack([cols0, cols1])
    partials = _sc_spmm(featT, vals, rows, cols)
    return _combine(partials, kernel, bias)


# speedup vs baseline: 17.2568x; 2.8301x over previous
import functools

import jax
import jax.numpy as jnp
from jax import lax
from jax.experimental import pallas as pl
from jax.experimental.pallas import tpu as pltpu
from jax.experimental.pallas import tpu_sc as plsc

_CH = 128


_NBUF = 4
_LEAD = 2
_ILEAD = 4
_IRN = 8


def _sc_spmm(featT, ind, vals):
    N, B = featT.shape
    S, nchunks_tot, _, _ = ind.shape
    n_sub = 16
    nchunks = nchunks_tot // n_sub
    rows_per_sub = N // n_sub

    mesh = plsc.VectorSubcoreMesh(core_axis_name="c", subcore_axis_name="s")

    @functools.partial(
        pl.kernel,
        mesh=mesh,
        compiler_params=pltpu.CompilerParams(use_tc_tiling_on_sc=False),
        out_type=jax.ShapeDtypeStruct((S, N, B), jnp.float32),
        scratch_types=[
            pltpu.VMEM((_NBUF, _CH, B), jnp.float32),
            pltpu.VMEM((_IRN, 2, _CH), jnp.int32),
            pltpu.VMEM((_IRN, _CH), jnp.float32),
            pltpu.VMEM_SHARED((N, B), jnp.float32),
            pltpu.SemaphoreType.DMA,
            pltpu.SemaphoreType.DMA,
            pltpu.SemaphoreType.DMA,
        ],
    )
    def k(featT_h, ind_h, val_h, out_h, gbufs, iring, vring, acc, isem, gsem,
          ssem):
        c = lax.axis_index("c")
        s = lax.axis_index("s")

        def issue_idx(q):
            slot = lax.rem(q, _IRN) if not isinstance(q, int) else q % _IRN
            pltpu.async_copy(ind_h.at[c, s * nchunks + q], iring.at[slot], isem)
            pltpu.async_copy(val_h.at[c, s * nchunks + q], vring.at[slot], isem)

        def wait_idx(q):
            slot = lax.rem(q, _IRN) if not isinstance(q, int) else q % _IRN
            pltpu.make_async_copy(
                ind_h.at[c, s * nchunks + q], iring.at[slot], isem).wait()
            pltpu.make_async_copy(
                val_h.at[c, s * nchunks + q], vring.at[slot], isem).wait()

        def start_gather(q):
            slot = lax.rem(q, _IRN) if not isinstance(q, int) else q % _IRN
            b = lax.rem(q, _NBUF) if not isinstance(q, int) else q % _NBUF
            pltpu.async_copy(featT_h.at[iring.at[slot, 0]], gbufs.at[b], gsem)

        def wait_gather(q):
            slot = lax.rem(q, _IRN) if not isinstance(q, int) else q % _IRN
            b = lax.rem(q, _NBUF) if not isinstance(q, int) else q % _NBUF
            pltpu.make_async_copy(
                featT_h.at[iring.at[slot, 0]], gbufs.at[b], gsem).wait()

        def start_scatter(q):
            slot = lax.rem(q, _IRN) if not isinstance(q, int) else q % _IRN
            b = lax.rem(q, _NBUF) if not isinstance(q, int) else q % _NBUF
            pltpu.async_copy(gbufs.at[b], acc.at[iring.at[slot, 1]], ssem,
                             add=True)

        def wait_scatter(q):
            slot = lax.rem(q, _IRN) if not isinstance(q, int) else q % _IRN
            b = lax.rem(q, _NBUF) if not isinstance(q, int) else q % _NBUF
            pltpu.make_async_copy(
                gbufs.at[b], acc.at[iring.at[slot, 1]], ssem).wait()

        def zrow(r, carry):
            for q in range(B // 16):
                gbufs[0, r, pl.ds(q * 16, 16)] = jnp.zeros((16,), jnp.float32)
            return carry

        lax.fori_loop(0, _CH, zrow, None)
        for kk in range(rows_per_sub // _CH):
            pltpu.sync_copy(gbufs.at[0],
                            acc.at[pl.ds(s * rows_per_sub + kk * _CH, _CH)])
        plsc.subcore_barrier()

        for q in range(_ILEAD):
            issue_idx(q)
        for q in range(_LEAD):
            wait_idx(q)
            start_gather(q)

        def chunk(q, carry):
            b = lax.rem(q, _NBUF)
            slot = lax.rem(q, _IRN)

            @pl.when(q + _ILEAD < nchunks)
            def _():
                issue_idx(q + _ILEAD)

            @pl.when(q >= _LEAD)
            def _():
                wait_scatter(q - _LEAD)

            @pl.when(q + _LEAD < nchunks)
            def _():
                wait_idx(q + _LEAD)
                start_gather(q + _LEAD)

            wait_gather(q)
            for j in range(_CH // 16):
                va = vring[slot, pl.ds(j * 16, 16)]
                for e in range(16):
                    r = j * 16 + e
                    v = va[e]
                    for u in range(B // 16):
                        sl = pl.ds(u * 16, 16)
                        gbufs[b, r, sl] = gbufs[b, r, sl] * v
            start_scatter(q)
            return carry

        lax.fori_loop(0, nchunks, chunk, None)
        for q in range(nchunks - _LEAD, nchunks):
            wait_scatter(q)
        plsc.subcore_barrier()
        pltpu.sync_copy(
            acc.at[pl.ds(s * rows_per_sub, rows_per_sub)],
            out_h.at[c, pl.ds(s * rows_per_sub, rows_per_sub)],
        )

    return k(featT, ind, vals)


def _combine(partials, weights, bias):
    S, N, B = partials.shape
    BN = 512

    def body(p_ref, k_ref, b_ref, o_ref):
        x = p_ref[0] * k_ref[0, 0] + p_ref[1] * k_ref[1, 0] + b_ref[0]
        o_ref[...] = jnp.clip(x, 0.0, 1.0).T

    return pl.pallas_call(
        body,
        grid=(N // BN,),
        in_specs=[
            pl.BlockSpec((S, BN, B), lambda i: (0, i, 0)),
            pl.BlockSpec(memory_space=pltpu.SMEM),
            pl.BlockSpec(memory_space=pltpu.SMEM),
        ],
        out_specs=pl.BlockSpec((B, BN), lambda i: (0, i)),
        out_shape=jax.ShapeDtypeStruct((B, N), jnp.float32),
    )(partials, weights, bias)


def kernel(features, vals0, vals1, kernel, bias, rows0, cols0, rows1, cols1):
    featT = features.T
    vals = jnp.stack([vals0, vals1]).reshape(2, -1, _CH)
    rows = jnp.stack([rows0, rows1]).reshape(2, -1, _CH)
    cols = jnp.stack([cols0, cols1]).reshape(2, -1, _CH)
    ind = jnp.stack([cols, rows], axis=2)
    partials = _sc_spmm(featT, ind, vals)
    return _combine(partials, kernel, bias)
